# baseline (speedup 1.0000x reference)
import functools

import jax
import jax.numpy as jnp
from jax import lax
from jax.experimental import pallas as pl
from jax.experimental.pallas import tpu as pltpu
from jax.experimental.pallas import tpu_sc as plsc

N = 50000
E = 800000
K = 5

NWORK = 32
EPT_A = 25088
EPAD = EPT_A * NWORK
CHUNK = 1792
GRP = 128
NCH_A = EPT_A // CHUNK
EPT_B = EPAD // 16
NCH_B = EPT_B // CHUNK
NGRP_CH = CHUNK // GRP

AROWS = 50048
NRNG = 6
R2 = 8336
TROWS = 41728
TRASH2 = R2 * 5

_i32 = jnp.int32
_f32 = jnp.float32


def _iota16():
    return lax.iota(_i32, 16)


def _full16(v):
    return jnp.full((16,), v, _i32)


def _edge_weights(av):
    u = jnp.minimum(jnp.maximum(av, 0.0), 1.0) * (K - 1)
    bot = jnp.minimum(u.astype(_i32), K - 2)
    frac = u - bot.astype(_f32)
    return bot, frac


def _sc_layer1(src_ref, dst_ref, attr_ref, x_ref, zeros_ref, s_out,
               s_sh, xv, srcb, dstb, attrb, dsts, botc, val):
    c = lax.axis_index("c")
    s = lax.axis_index("s")
    tid = s
    wid = c * 16 + s

    pltpu.sync_copy(zeros_ref.at[pl.ds(tid * 3128, 3128), :],
                    s_sh.at[pl.ds(tid * 3128, 3128), :])
    pltpu.sync_copy(x_ref, xv)
    pltpu.sync_copy(zeros_ref.at[pl.ds(0, GRP), :], val)
    plsc.subcore_barrier()

    iota = _iota16()
    z16 = jnp.zeros((16,), _i32)
    zf16 = jnp.zeros((16,), _f32)
    ones = jnp.ones((16,), _f32)
    col5 = _full16(5)
    e0 = wid * EPT_A

    def chunk_body(ch, _):
        base = e0 + ch * CHUNK
        pltpu.sync_copy(src_ref.at[pl.ds(base, CHUNK)], srcb)
        pltpu.sync_copy(dst_ref.at[pl.ds(base, CHUNK)], dstb)
        pltpu.sync_copy(attr_ref.at[pl.ds(base, CHUNK)], attrb)

        def grp_body(g, _):
            gbase = g * GRP
            for i in range(8):
                o = gbase + 16 * i
                pos = iota + 16 * i
                sv = srcb[pl.ds(o, 16)]
                dv = dstb[pl.ds(o, 16)]
                av = attrb[pl.ds(o, 16)]
                bot, frac = _edge_weights(av)
                xg = plsc.load_gather(xv, [sv])
                plsc.store_scatter(dsts, [z16, pos], dv)
                botc[pl.ds(16 * i, 16)] = bot
                plsc.store_scatter(val, [pos, bot], (1.0 - frac) * xg)
                plsc.store_scatter(val, [pos, bot + 1], frac * xg)
                plsc.store_scatter(val, [pos, col5], ones)
            pltpu.sync_copy(val, s_sh.at[dsts.at[0]], add=True)
            for i in range(8):
                pos = iota + 16 * i
                bot = botc[pl.ds(16 * i, 16)]
                plsc.store_scatter(val, [pos, bot], zf16)
                plsc.store_scatter(val, [pos, bot + 1], zf16)
                plsc.store_scatter(val, [pos, col5], zf16)
            return _

        return lax.fori_loop(0, NGRP_CH, grp_body, _)

    lax.fori_loop(0, NCH_A, chunk_body, 0)
    plsc.subcore_barrier()
    pltpu.sync_copy(s_sh.at[pl.ds(tid * 3128, 3128), :],
                    s_out.at[c].at[pl.ds(tid * 3128, 3128), :])


def _run_layer1(src, dst, attr, x_flat, zeros_a):
    mesh = plsc.VectorSubcoreMesh(core_axis_name="c", subcore_axis_name="s")
    f = functools.partial(
        pl.kernel,
        out_type=jax.ShapeDtypeStruct((2, AROWS, 16), _f32),
        mesh=mesh,
        scratch_types=[
            pltpu.VMEM_SHARED((AROWS, 16), _f32),
            pltpu.VMEM((N,), _f32),
            pltpu.VMEM((CHUNK,), _i32),
            pltpu.VMEM((CHUNK,), _i32),
            pltpu.VMEM((CHUNK,), _f32),
            pltpu.VMEM((1, GRP), _i32),
            pltpu.VMEM((GRP,), _i32),
            pltpu.VMEM((GRP, 16), _f32),
        ],
        compiler_params=pltpu.CompilerParams(needs_layout_passes=False, use_tc_tiling_on_sc=False),
    )(_sc_layer1)
    return f(src, dst, attr, x_flat, zeros_a)


def _sc_layer2(src_ref, dst_ref, attr_ref, h_ref, zeros_ref, t_out,
               t_sh, srcb, dstb, attrb, cs_src, cs_frac, cs_idx,
               idx0s, idx1s, hrows, val0, val1):
    c = lax.axis_index("c")
    s = lax.axis_index("s")
    tid = s
    iota = _iota16()
    z16 = jnp.zeros((16,), _i32)
    e0 = tid * EPT_B

    for p in range(3):
        rng = 3 * c + p
        nbase = rng * R2

        pltpu.sync_copy(zeros_ref.at[pl.ds(tid * 2608, 2608), :],
                        t_sh.at[pl.ds(tid * 2608, 2608), :])
        plsc.subcore_barrier()

        def chunk_body(ch, _):
            base = e0 + ch * CHUNK
            pltpu.sync_copy(src_ref.at[pl.ds(base, CHUNK)], srcb)
            pltpu.sync_copy(dst_ref.at[pl.ds(base, CHUNK)], dstb)
            pltpu.sync_copy(attr_ref.at[pl.ds(base, CHUNK)], attrb)

            def compress_body(i, cnt):
                o = 16 * i
                sv = srcb[pl.ds(o, 16)]
                dv = dstb[pl.ds(o, 16)]
                av = attrb[pl.ds(o, 16)]
                bot, frac = _edge_weights(av)
                m = jnp.logical_and(dv >= nbase, dv < nbase + R2)
                idx0 = jnp.where(m, (dv - nbase) * K + bot, TRASH2)
                plsc.store_compressed(cs_src.at[pl.ds(cnt, 16)], sv, mask=m)
                plsc.store_compressed(cs_frac.at[pl.ds(cnt, 16)], frac, mask=m)
                plsc.store_compressed(cs_idx.at[pl.ds(cnt, 16)], idx0, mask=m)
                return cnt + jnp.max(plsc.all_reduce_population_count(m))

            cnt = lax.fori_loop(0, CHUNK // 16, compress_body, jnp.int32(0))
            for j in range(8):
                cs_src[pl.ds(cnt + 16 * j, 16)] = z16
                cs_frac[pl.ds(cnt + 16 * j, 16)] = jnp.zeros((16,), _f32)
                cs_idx[pl.ds(cnt + 16 * j, 16)] = _full16(TRASH2)
            ngrp = (cnt + GRP - 1) // GRP

            def grp_body(g, _):
                gbase = g * GRP
                pltpu.sync_copy(h_ref.at[cs_src.at[pl.ds(gbase, GRP)]], hrows)
                for i in range(8):
                    pos = iota + 16 * i
                    v = cs_idx[pl.ds(gbase + 16 * i, 16)]
                    plsc.store_scatter(idx0s, [z16, pos], v)
                    plsc.store_scatter(idx1s, [z16, pos], v + 1)

                def scale_body(j, _):
                    f = plsc.load_gather(cs_frac, [_full16(gbase + j)])
                    jj = _full16(j)
                    r0 = plsc.load_gather(hrows, [jj, iota])
                    r1 = plsc.load_gather(hrows, [jj, iota + 16])
                    plsc.store_scatter(val0, [jj, iota], (1.0 - f) * r0)
                    plsc.store_scatter(val0, [jj, iota + 16], (1.0 - f) * r1)
                    plsc.store_scatter(val1, [jj, iota], f * r0)
                    plsc.store_scatter(val1, [jj, iota + 16], f * r1)
                    return _

                lax.fori_loop(0, GRP, scale_body, 0)
                pltpu.sync_copy(val0, t_sh.at[idx0s.at[0]], add=True)
                pltpu.sync_copy(val1, t_sh.at[idx1s.at[0]], add=True)
                return _

            return lax.fori_loop(0, ngrp, grp_body, _)

        lax.fori_loop(0, NCH_B, chunk_body, 0)
        plsc.subcore_barrier()
        pltpu.sync_copy(t_sh.at[pl.ds(tid * 2608, 2608), :],
                        t_out.at[rng].at[pl.ds(tid * 2608, 2608), :])
        plsc.subcore_barrier()


def _run_layer2(src, dst, attr, h, zeros_b):
    mesh = plsc.VectorSubcoreMesh(core_axis_name="c", subcore_axis_name="s")
    f = functools.partial(
        pl.kernel,
        out_type=jax.ShapeDtypeStruct((NRNG, TROWS, 32), _f32),
        mesh=mesh,
        scratch_types=[
            pltpu.VMEM_SHARED((TROWS, 32), _f32),
            pltpu.VMEM((CHUNK,), _i32),
            pltpu.VMEM((CHUNK,), _i32),
            pltpu.VMEM((CHUNK,), _f32),
            pltpu.VMEM((CHUNK + GRP,), _i32),
            pltpu.VMEM((CHUNK + GRP,), _f32),
            pltpu.VMEM((CHUNK + GRP,), _i32),
            pltpu.VMEM((1, GRP), _i32),
            pltpu.VMEM((1, GRP), _i32),
            pltpu.VMEM((GRP, 32), _f32),
            pltpu.VMEM((GRP, 32), _f32),
            pltpu.VMEM((GRP, 32), _f32),
        ],
        compiler_params=pltpu.CompilerParams(needs_layout_passes=False, use_tc_tiling_on_sc=False),
    )(_sc_layer2)
    return f(src, dst, attr, h, zeros_b)


def _elu(v):
    return jnp.where(v > 0, v, jnp.exp(v) - 1.0)


def _dense1_body(s_ref, x_ref, w_ref, r_ref, b_ref, h_ref, deg_ref):
    sblk = s_ref[0] + s_ref[1]
    deg = jnp.maximum(sblk[:, 5:6], 1.0)
    acc = jnp.dot(sblk[:, 0:5], w_ref[...], preferred_element_type=_f32)
    out = acc / deg + x_ref[...] * r_ref[...] + b_ref[...]
    h_ref[...] = _elu(out)
    deg_ref[...] = deg


def _run_dense1(s, x, w1r, root1, b1):
    bn = 1000
    return pl.pallas_call(
        _dense1_body,
        out_shape=[jax.ShapeDtypeStruct((N, 32), _f32),
                   jax.ShapeDtypeStruct((N, 1), _f32)],
        grid=(N // bn,),
        in_specs=[
            pl.BlockSpec((2, bn, 16), lambda i: (0, i, 0)),
            pl.BlockSpec((bn, 1), lambda i: (i, 0)),
            pl.BlockSpec((5, 32), lambda i: (0, 0)),
            pl.BlockSpec((1, 32), lambda i: (0, 0)),
            pl.BlockSpec((1, 32), lambda i: (0, 0)),
        ],
        out_specs=[pl.BlockSpec((bn, 32), lambda i: (i, 0)),
                   pl.BlockSpec((bn, 1), lambda i: (i, 0))],
    )(s, x, w1r, root1, b1)


def _dense2_body(t_ref, h_ref, deg_ref, w2_ref, r2_ref, b2_ref,
                 l1w_ref, l1b_ref, l2w_ref, l2b_ref, o_ref):
    acc = jnp.dot(t_ref[...], w2_ref[...], preferred_element_type=_f32)
    out2 = acc / deg_ref[...] + jnp.dot(h_ref[...], r2_ref[...],
                                        preferred_element_type=_f32) + b2_ref[...]
    h2 = _elu(out2)
    h3 = _elu(jnp.dot(h2, l1w_ref[...], preferred_element_type=_f32) + l1b_ref[...])
    lg = jnp.dot(h3, l2w_ref[...], preferred_element_type=_f32) + l2b_ref[...]
    m = jnp.max(lg, axis=1, keepdims=True)
    o_ref[...] = lg - m - jnp.log(jnp.sum(jnp.exp(lg - m), axis=1, keepdims=True))


def _run_dense2(t, h, deg, w2r, root2, b2, l1w, l1b, l2w, l2b):
    bn = 1000
    return pl.pallas_call(
        _dense2_body,
        out_shape=jax.ShapeDtypeStruct((N, 10), _f32),
        grid=(N // bn,),
        in_specs=[
            pl.BlockSpec((bn, 160), lambda i: (i, 0)),
            pl.BlockSpec((bn, 32), lambda i: (i, 0)),
            pl.BlockSpec((bn, 1), lambda i: (i, 0)),
            pl.BlockSpec((160, 64), lambda i: (0, 0)),
            pl.BlockSpec((32, 64), lambda i: (0, 0)),
            pl.BlockSpec((1, 64), lambda i: (0, 0)),
            pl.BlockSpec((64, 128), lambda i: (0, 0)),
            pl.BlockSpec((1, 128), lambda i: (0, 0)),
            pl.BlockSpec((128, 10), lambda i: (0, 0)),
            pl.BlockSpec((1, 10), lambda i: (0, 0)),
        ],
        out_specs=pl.BlockSpec((bn, 10), lambda i: (i, 0)),
    )(t, h, deg, w2r, root2, b2, l1w, l1b, l2w, l2b)


def kernel(x, edge_index, edge_attr, W1, root1, b1, W2, root2, b2,
           l1w, l1b, l2w, l2b):
    src = edge_index[0].astype(_i32)
    dst = edge_index[1].astype(_i32)
    attr = edge_attr[:, 0].astype(_f32)
    npad = EPAD - E
    src = jnp.concatenate([src, jnp.zeros((npad,), _i32)])
    dst = jnp.concatenate([dst, jnp.full((npad,), N, _i32)])
    attr = jnp.concatenate([attr, jnp.zeros((npad,), _f32)])

    zeros_a = jnp.zeros((AROWS, 16), _f32)
    zeros_b = jnp.zeros((TROWS, 32), _f32)

    s = _run_layer1(src, dst, attr, x[:, 0], zeros_a)
    s = s[:, :N, :]
    h, deg = _run_dense1(s, x, W1.reshape(K, 32), root1, b1.reshape(1, 32))

    t = _run_layer2(src, dst, attr, h, zeros_b)
    t = t[:, :TRASH2, :].reshape(NRNG * R2, K * 32)[:N]
    out = _run_dense2(t, h, deg, W2.reshape(K * 32, 64), root2,
                      b2.reshape(1, 64), l1w, l1b.reshape(1, 128),
                      l2w, l2b.reshape(1, 10))
    return out
